# speedup vs baseline: 56.0209x; 56.0209x over previous
import jax
import jax.numpy as jnp
from jax.experimental import pallas as pl
from jax.experimental.pallas import tpu as pltpu

_SIGMA = 0.5
_THRESH = 0.2
_N = 10000
_ROWS = 80
_LANES = 128
_PAD = _ROWS * _LANES
_CHUNK = 64
_NEG = -jnp.inf


def _nms_kernel(x1_ref, y1_ref, x2_ref, y2_ref, s_ref, out_ref, w_ref, a_ref):
    x1 = x1_ref[...]
    y1 = y1_ref[...]
    x2 = x2_ref[...]
    y2 = y2_ref[...]
    a_ref[...] = (x2 - x1) * (y2 - y1)
    s = s_ref[...]
    w_ref[...] = jnp.where(s > _THRESH, s, _NEG)
    out_ref[...] = jnp.zeros_like(s)

    rows = jax.lax.broadcasted_iota(jnp.int32, (_ROWS, _LANES), 0)
    cols = jax.lax.broadcasted_iota(jnp.int32, (_ROWS, _LANES), 1)
    flat = rows * _LANES + cols

    def pick_once(_, carry):
        w = w_ref[...]
        m = jnp.max(w, axis=(0, 1), keepdims=True)
        eq = w == m
        cand = jnp.where(eq, flat, jnp.int32(2**30))
        idx = jnp.min(cand, axis=(0, 1), keepdims=True)
        onehot = flat == idx
        ohf = jnp.where(onehot, 1.0, 0.0)
        bx1 = jnp.sum(x1 * ohf, axis=(0, 1), keepdims=True)
        by1 = jnp.sum(y1 * ohf, axis=(0, 1), keepdims=True)
        bx2 = jnp.sum(x2 * ohf, axis=(0, 1), keepdims=True)
        by2 = jnp.sum(y2 * ohf, axis=(0, 1), keepdims=True)
        area = a_ref[...]
        ba = jnp.sum(area * ohf, axis=(0, 1), keepdims=True)
        ix1 = jnp.maximum(bx1, x1)
        iy1 = jnp.maximum(by1, y1)
        ix2 = jnp.minimum(bx2, x2)
        iy2 = jnp.minimum(by2, y2)
        inter = jnp.maximum(ix2 - ix1, 0.0) * jnp.maximum(iy2 - iy1, 0.0)
        iou = inter / (ba + area - inter + 1e-9)
        decay = jnp.exp(-(iou * iou) / _SIGMA)
        neww = jnp.where(onehot, _NEG, w * decay)
        w_ref[...] = neww
        rec = onehot & (m > _THRESH)
        out_ref[...] = jnp.where(rec, m, out_ref[...])
        return carry

    def cond(c):
        i, live = c
        return live & (i < _PAD // _CHUNK + 2)

    def body(c):
        i, _ = c
        jax.lax.fori_loop(0, _CHUNK, pick_once, 0, unroll=False)
        return (i + 1, jnp.max(w_ref[...]) > _THRESH)

    jax.lax.while_loop(cond, body, (jnp.int32(0), jnp.bool_(True)))


@jax.jit
def kernel(results1, results2):
    results = jnp.concatenate([results1, results2], axis=0)
    box = results[:, 2:6]
    scores = results[:, 13]

    def col(v, fill):
        return jnp.pad(v, (0, _PAD - _N), constant_values=fill).reshape(
            _ROWS, _LANES
        )

    x1 = col(box[:, 0], 0.0)
    y1 = col(box[:, 1], 0.0)
    x2 = col(box[:, 2], 0.0)
    y2 = col(box[:, 3], 0.0)
    s = col(scores, 0.0)

    final2d = pl.pallas_call(
        _nms_kernel,
        out_shape=jax.ShapeDtypeStruct((_ROWS, _LANES), jnp.float32),
        scratch_shapes=[
            pltpu.VMEM((_ROWS, _LANES), jnp.float32),
            pltpu.VMEM((_ROWS, _LANES), jnp.float32),
        ],
    )(x1, y1, x2, y2, s)

    final = final2d.reshape(_PAD)[:_N]
    keep = final > _THRESH
    out = results.at[:, 13].set(final)
    out = jnp.where(keep[:, None], out, 0.0)
    return out


# speedup vs baseline: 82.7219x; 1.4766x over previous
import jax
import jax.numpy as jnp
from jax.experimental import pallas as pl
from jax.experimental.pallas import tpu as pltpu

_SIGMA = 0.5
_THRESH = 0.2
_N = 10000
_ROWS = 80
_LANES = 128
_PAD = _ROWS * _LANES
_K = 8
_NEG = -jnp.inf


def _nms_kernel(x1_ref, y1_ref, x2_ref, y2_ref, s_ref, out_ref, v_ref):
    x1 = x1_ref[...]
    y1 = y1_ref[...]
    x2 = x2_ref[...]
    y2 = y2_ref[...]
    area = (x2 - x1) * (y2 - y1)
    s = s_ref[...]
    v_ref[...] = jnp.where(s > _THRESH, s, 0.0)

    rows = jax.lax.broadcasted_iota(jnp.int32, (_ROWS, _LANES), 0)
    cols = jax.lax.broadcasted_iota(jnp.int32, (_ROWS, _LANES), 1)
    flat = rows * _LANES + cols

    def extract(arr, oh):
        return jnp.sum(jnp.where(oh, arr, 0.0), axis=(0, 1), keepdims=True)

    def body(c):
        i, _ = c
        v = v_ref[...]

        cum = v
        idxs, ms, bxs = [], [], []
        for _j in range(_K):
            idx = jnp.argmax(cum)
            oh = flat == idx
            m = extract(cum, oh)
            cum = jnp.where(oh, _NEG, cum)
            idxs.append(idx)
            ms.append(m)
            bxs.append(
                (
                    extract(x1, oh),
                    extract(y1, oh),
                    extract(x2, oh),
                    extract(y2, oh),
                    extract(area, oh),
                )
            )

        vals = []
        val = None
        for j in range(_K):
            ok = None
            aj = bxs[j]
            for i2 in range(j):
                ai = bxs[i2]
                ix = jnp.minimum(aj[2], ai[2]) - jnp.maximum(aj[0], ai[0])
                iy = jnp.minimum(aj[3], ai[3]) - jnp.maximum(aj[1], ai[1])
                ov = (ix > 0.0) & (iy > 0.0)
                ok = ov if ok is None else (ok | ov)
            nov = jnp.ones_like(ms[j], dtype=jnp.bool_) if ok is None else ~ok
            val = nov if val is None else (val & nov)
            vals.append(val)

        newv = v
        for j in range(_K):
            bx1, by1, bx2, by2, ba = bxs[j]
            ix1 = jnp.maximum(bx1, x1)
            iy1 = jnp.maximum(by1, y1)
            ix2 = jnp.minimum(bx2, x2)
            iy2 = jnp.minimum(by2, y2)
            inter = jnp.maximum(ix2 - ix1, 0.0) * jnp.maximum(iy2 - iy1, 0.0)
            iou = inter / (ba + area - inter + 1e-9)
            e = jnp.where(vals[j], jnp.exp(-(iou * iou) / _SIGMA), 1.0)
            newv = newv * e
        newv = jnp.where(v > 0.0, newv, v)

        for j in range(_K):
            rec = (flat == idxs[j]) & vals[j] & (ms[j] > _THRESH)
            newv = jnp.where(rec, -ms[j], newv)

        v_ref[...] = newv
        return (i + 1, jnp.max(newv) > _THRESH)

    def cond(c):
        i, live = c
        return live & (i < _PAD)

    jax.lax.while_loop(cond, body, (jnp.int32(0), jnp.bool_(True)))
    out_ref[...] = v_ref[...]


@jax.jit
def kernel(results1, results2):
    results = jnp.concatenate([results1, results2], axis=0)
    box = results[:, 2:6]
    scores = results[:, 13]

    def col(vv):
        return jnp.pad(vv, (0, _PAD - _N)).reshape(_ROWS, _LANES)

    final2d = pl.pallas_call(
        _nms_kernel,
        out_shape=jax.ShapeDtypeStruct((_ROWS, _LANES), jnp.float32),
        scratch_shapes=[pltpu.VMEM((_ROWS, _LANES), jnp.float32)],
    )(col(box[:, 0]), col(box[:, 1]), col(box[:, 2]), col(box[:, 3]), col(scores))

    vflat = final2d.reshape(_PAD)[:_N]
    keep = vflat < -_THRESH
    out = results.at[:, 13].set(-vflat)
    out = jnp.where(keep[:, None], out, 0.0)
    return out
